# baseline (speedup 1.0000x reference)
import functools

import jax
import jax.numpy as jnp
from jax import lax
from jax.experimental import pallas as pl
from jax.experimental.pallas import tpu as pltpu
from jax.experimental.pallas import tpu_sc as plsc

N_CLASSES = 100000
N_FEAT = 128
BATCH = 8

SC_N = 20480
TC_N = N_CLASSES - SC_N

BLOCK_C = 8192
NSTREAM = 2
_STEP_C = NSTREAM * BLOCK_C
_TC_GRID = -(-TC_N // _STEP_C)
_TC_LAST_VALID = (TC_N - 1) // BLOCK_C

NWORKERS = 32
CHUNK = SC_N // NWORKERS
CTILES = 4
_CBLOCKS = CHUNK // (16 * CTILES)


def _tc_body(x_ref, b_ref, *rest):
    w_refs = rest[:NSTREAM]
    o_ref = rest[NSTREAM]
    x = x_ref[...]
    for s in range(NSTREAM):
        acc = jax.lax.dot_general(
            x, w_refs[s][...],
            dimension_numbers=(((1,), (1,)), ((), ())),
            preferred_element_type=jnp.float32,
        )
        sl = pl.ds(s * BLOCK_C, BLOCK_C)
        o_ref[:, sl] = acc + b_ref[:, sl]


def _w_index_map(s):
    def index_map(i):
        return (jnp.minimum(i * NSTREAM + s, _TC_LAST_VALID), 0)
    return index_map


def _tc_head(x, weight, bias2d):
    in_specs = [
        pl.BlockSpec((BATCH, N_FEAT), lambda i: (0, 0)),
        pl.BlockSpec((1, _STEP_C), lambda i: (0, i)),
    ] + [
        pl.BlockSpec((BLOCK_C, N_FEAT), _w_index_map(s)) for s in range(NSTREAM)
    ]
    return pl.pallas_call(
        _tc_body,
        grid=(_TC_GRID,),
        in_specs=in_specs,
        out_specs=pl.BlockSpec((BATCH, _STEP_C), lambda i: (0, i)),
        out_shape=jax.ShapeDtypeStruct((BATCH, TC_N), jnp.float32),
        compiler_params=pltpu.CompilerParams(
            dimension_semantics=("parallel",),
        ),
    )(x, bias2d, *([weight] * NSTREAM))


_RB = 16 * CTILES


def _sc_body(w_hbm, xs_hbm, b_hbm, o_hbm, wbuf, xsbuf, obuf, bbuf, sems):
    wid = lax.axis_index("s") * 2 + lax.axis_index("c")
    base = wid * CHUNK

    def wcopy(blk, slot):
        return pltpu.make_async_copy(
            w_hbm.at[pl.ds(TC_N + base + blk * _RB, _RB)],
            wbuf.at[slot], sems.at[slot])

    wcopy(0, 0).start()
    pltpu.sync_copy(xs_hbm, xsbuf)
    pltpu.sync_copy(b_hbm.at[pl.ds(TC_N + base, CHUNK)], bbuf)

    lane = lax.iota(jnp.int32, 16)
    rows = [t * 16 + lane for t in range(CTILES)]

    for blk in range(_CBLOCKS):
        slot = blk % 2
        if blk + 1 < _CBLOCKS:
            wcopy(blk + 1, 1 - slot).start()
        wcopy(blk, slot).wait()
        wslab = wbuf.at[slot]

        def k_body(k, accs, wslab=wslab):
            kk = jnp.full((16,), k, jnp.int32)
            cols = [
                plsc.load_gather(wslab, [rows[t], kk]) for t in range(CTILES)
            ]
            new = []
            for b in range(BATCH):
                xs = xsbuf[k, pl.ds(b * 16, 16)]
                for t in range(CTILES):
                    new.append(accs[b * CTILES + t] + cols[t] * xs)
            return tuple(new)

        zeros = tuple(
            jnp.zeros((16,), jnp.float32) for _ in range(BATCH * CTILES))
        accs = lax.fori_loop(0, N_FEAT, k_body, zeros)
        cbase = blk * _RB
        for b in range(BATCH):
            for t in range(CTILES):
                sl = pl.ds(cbase + t * 16, 16)
                obuf[b, sl] = accs[b * CTILES + t] + bbuf[sl]

    for b in range(BATCH):
        pltpu.sync_copy(obuf.at[b], o_hbm.at[b, pl.ds(base, CHUNK)])


_sc_head = functools.partial(
    pl.kernel,
    out_type=jax.ShapeDtypeStruct((BATCH, SC_N), jnp.float32),
    mesh=plsc.VectorSubcoreMesh(core_axis_name="c", subcore_axis_name="s"),
    scratch_types=[
        pltpu.VMEM((2, _RB, N_FEAT), jnp.float32),
        pltpu.VMEM((N_FEAT, BATCH * 16), jnp.float32),
        pltpu.VMEM((BATCH, CHUNK), jnp.float32),
        pltpu.VMEM((CHUNK,), jnp.float32),
        pltpu.SemaphoreType.DMA((2,)),
    ],
    compiler_params=pltpu.CompilerParams(needs_layout_passes=False),
)(_sc_body)


def kernel(x, weight, bias):
    bias2d = bias.reshape(1, N_CLASSES)
    x_splat = jnp.broadcast_to(
        x.T[:, :, None], (N_FEAT, BATCH, 16)).reshape(N_FEAT, BATCH * 16)
    tc_out = _tc_head(x, weight, bias2d)
    sc_out = _sc_head(weight, x_splat, bias)
    return jnp.concatenate([tc_out, sc_out], axis=1)
